# baseline (speedup 1.0000x reference)
import functools

import jax
import jax.numpy as jnp
from jax import lax
from jax.experimental import pallas as pl
from jax.experimental.pallas import tpu as pltpu
from jax.experimental.pallas import tpu_sc as plsc

NUM_EMB = 1_000_000
DIM = 32
BATCH = 16384
HIST = 50

NC, NS = 2, 16
D_PER_SC = DIM // NC
B_PER_T = BATCH // NS
HGRP = 5
NGRP = HIST // HGRP
STAGERS = 8
STAGE_N = NUM_EMB // STAGERS


def _sc_gather(tok_t, tab_t):
    mesh = plsc.VectorSubcoreMesh(
        core_axis_name="c", subcore_axis_name="s", num_cores=NC, num_subcores=NS
    )

    @functools.partial(
        pl.kernel,
        mesh=mesh,
        out_type=jax.ShapeDtypeStruct((HIST, DIM, BATCH), jnp.float32),
        scratch_types=[
            pltpu.VMEM((HIST, B_PER_T), jnp.int32),
            pltpu.VMEM((2, HGRP, B_PER_T), jnp.float32),
            pltpu.VMEM_SHARED((NUM_EMB,), jnp.float32),
            pltpu.SemaphoreType.DMA((2,)),
            pltpu.SemaphoreType.DMA((2,)),
        ],
        compiler_params=pltpu.CompilerParams(use_tc_tiling_on_sc=False),
    )
    def k(tok_hbm, tab_hbm, out_hbm, idx_v, out_v, row_sp, gsem, ssem):
        c = lax.axis_index("c")
        t = lax.axis_index("s")
        b0 = t * B_PER_T

        pltpu.sync_copy(tok_hbm.at[:, pl.ds(b0, B_PER_T)], idx_v)

        def fire_gathers(grp, q):
            h0 = grp * HGRP
            for hh in range(HGRP):
                pltpu.async_copy(
                    row_sp.at[idx_v.at[h0 + hh]],
                    out_v.at[q, hh],
                    gsem.at[q],
                )

        def wait_gathers(grp, q):
            h0 = grp * HGRP
            for hh in range(HGRP):
                pltpu.make_async_copy(
                    row_sp.at[idx_v.at[h0 + hh]],
                    out_v.at[q, hh],
                    gsem.at[q],
                ).wait()

        def fire_store(grp, q, dg):
            pltpu.async_copy(
                out_v.at[q],
                out_hbm.at[pl.ds(grp * HGRP, HGRP), dg, pl.ds(b0, B_PER_T)],
                ssem.at[q],
            )

        def wait_store(grp, q, dg):
            pltpu.make_async_copy(
                out_v.at[q],
                out_hbm.at[pl.ds(grp * HGRP, HGRP), dg, pl.ds(b0, B_PER_T)],
                ssem.at[q],
            ).wait()

        def body(dl, _):
            dg = c * D_PER_SC + dl

            plsc.subcore_barrier()

            @pl.when(t < STAGERS)
            def _():
                off = t * STAGE_N
                pltpu.sync_copy(
                    tab_hbm.at[dg, pl.ds(off, STAGE_N)],
                    row_sp.at[pl.ds(off, STAGE_N)],
                )

            plsc.subcore_barrier()

            def unit(grp):
                u = dl * NGRP + grp
                q = grp % 2

                @pl.when(u >= 2)
                def _():
                    prev = u - 2
                    pg = prev % NGRP
                    pd = c * D_PER_SC + prev // NGRP
                    wait_store(pg, q, pd)

                fire_gathers(grp, q)

            unit(0)
            for grp in range(NGRP):
                if grp + 1 < NGRP:
                    unit(grp + 1)
                wait_gathers(grp, grp % 2)
                fire_store(grp, grp % 2, dg)

            return ()

        lax.fori_loop(0, D_PER_SC, body, (), unroll=False)

        last = c * D_PER_SC + D_PER_SC - 1
        for grp in range(NGRP - 2, NGRP):
            wait_store(grp, grp % 2, last)

    return k(tok_t, tab_t)


def kernel(tokens, emb_table):
    tok_t = tokens.T.astype(jnp.int32)
    tab_t = emb_table.T
    out_phys = _sc_gather(tok_t, tab_t)
    return jnp.transpose(out_phys, (2, 0, 1))


# speedup vs baseline: 2.4460x; 2.4460x over previous
import functools

import jax
import jax.numpy as jnp
from jax import lax
from jax.experimental import pallas as pl
from jax.experimental.pallas import tpu as pltpu
from jax.experimental.pallas import tpu_sc as plsc

NUM_EMB = 1_000_000
DIM = 32
BATCH = 16384
HIST = 50

NC, NS = 2, 16
NW = NC * NS
BW = BATCH // NW
CB_N = BW // 128
NRING = 4


def _sc_gather(tok_t, tab):
    mesh = plsc.VectorSubcoreMesh(
        core_axis_name="c", subcore_axis_name="s", num_cores=NC, num_subcores=NS
    )

    @functools.partial(
        pl.kernel,
        mesh=mesh,
        out_type=jax.ShapeDtypeStruct((HIST, DIM // 8, BATCH // 128, 8, 128),
                                      jnp.float32),
        scratch_types=[
            pltpu.VMEM((HIST, BW), jnp.int32),
            pltpu.VMEM((NRING, 128, DIM), jnp.float32),
            pltpu.VMEM((NRING, DIM // 8, 8, 128), jnp.float32),
            pltpu.SemaphoreType.DMA((NRING,)),
            pltpu.SemaphoreType.DMA((NRING,)),
        ],
        compiler_params=pltpu.CompilerParams(
            use_tc_tiling_on_sc=False, needs_layout_passes=False
        ),
    )
    def k(tok_hbm, tab_hbm, out_hbm, idx_v, g_v, s_v, gsem, ssem):
        c = lax.axis_index("c")
        t = lax.axis_index("s")
        w = t * NC + c
        b0 = w * BW
        bc0 = w * CB_N

        pltpu.sync_copy(tok_hbm.at[:, pl.ds(b0, BW)], idx_v)

        rows16 = [lax.iota(jnp.int32, 16) + 16 * j2 for j2 in range(8)]

        def gather_copy(h, cb):
            r = (cb % NRING)
            return pltpu.make_async_copy(
                tab_hbm.at[idx_v.at[h, pl.ds(cb * 128, 128)]],
                g_v.at[r],
                gsem.at[r],
            )

        def store_copy(h, cb):
            r = cb % NRING
            return pltpu.make_async_copy(
                s_v.at[r],
                out_hbm.at[h, :, bc0 + cb, :, :],
                ssem.at[r],
            )

        def fire_gather(h, cb):
            pltpu.async_copy(
                tab_hbm.at[idx_v.at[h, pl.ds(cb * 128, 128)]],
                g_v.at[cb % NRING],
                gsem.at[cb % NRING],
            )

        def transpose_chunk(r):
            for dd in range(DIM // 8):
                for ds_ in range(8):
                    d = dd * 8 + ds_
                    cols = jnp.full((16,), d, jnp.int32)
                    for j2 in range(8):
                        v = plsc.load_gather(g_v.at[r], [rows16[j2], cols])
                        s_v[r, dd, ds_, pl.ds(16 * j2, 16)] = v

        for cb in range(NRING):
            fire_gather(0, cb)

        def body(h, _):
            for cb in range(CB_N):
                r = cb % NRING
                gather_copy(h, cb).wait()

                @pl.when(h >= 1)
                def _():
                    store_copy(h - 1, cb).wait()

                transpose_chunk(r)

                @pl.when(h + 1 < HIST)
                def _():
                    fire_gather(h + 1, cb)

                pltpu.async_copy(
                    s_v.at[r],
                    out_hbm.at[h, :, bc0 + cb, :, :],
                    ssem.at[r],
                )

            return ()

        lax.fori_loop(0, HIST, body, (), unroll=False)

        for cb in range(CB_N):
            store_copy(HIST - 1, cb).wait()

    return k(tok_t, tab)


def kernel(tokens, emb_table):
    tok_t = tokens.T.astype(jnp.int32)
    out5 = _sc_gather(tok_t, emb_table)
    out = jnp.transpose(out5, (2, 4, 0, 1, 3))
    return out.reshape(BATCH, HIST, DIM)


# speedup vs baseline: 2.6751x; 1.0937x over previous
import functools

import jax
import jax.numpy as jnp
from jax import lax
from jax.experimental import pallas as pl
from jax.experimental.pallas import tpu as pltpu
from jax.experimental.pallas import tpu_sc as plsc

NUM_EMB = 1_000_000
DIM = 32
BATCH = 16384
HIST = 50

NC, NS = 2, 16
NW = NC * NS
BW = BATCH // NW
CB_N = BW // 128
NRING = 4


def _sc_gather(tok_t, tab):
    mesh = plsc.VectorSubcoreMesh(
        core_axis_name="c", subcore_axis_name="s", num_cores=NC, num_subcores=NS
    )

    @functools.partial(
        pl.kernel,
        mesh=mesh,
        out_type=jax.ShapeDtypeStruct((HIST, DIM // 8, BATCH // 128, 8, 128),
                                      jnp.float32),
        scratch_types=[
            pltpu.VMEM((HIST, BW), jnp.int32),
            pltpu.VMEM((NRING, 128, DIM), jnp.float32),
            pltpu.VMEM((NRING, DIM // 8, 8, 128), jnp.float32),
            pltpu.SemaphoreType.DMA((NRING,)),
            pltpu.SemaphoreType.DMA((NRING,)),
        ],
        compiler_params=pltpu.CompilerParams(
            use_tc_tiling_on_sc=False, needs_layout_passes=False
        ),
    )
    def k(tok_hbm, tab_hbm, out_hbm, idx_v, g_v, s_v, gsem, ssem):
        c = lax.axis_index("c")
        t = lax.axis_index("s")
        w = t * NC + c
        b0 = w * BW
        bc0 = w * CB_N

        pltpu.sync_copy(tok_hbm.at[:, pl.ds(b0, BW)], idx_v)

        rows16 = [lax.iota(jnp.int32, 16) + 16 * j2 for j2 in range(8)]

        def gather_copy(h, cb):
            r = (cb % NRING)
            return pltpu.make_async_copy(
                tab_hbm.at[idx_v.at[h, pl.ds(cb * 128, 128)]],
                g_v.at[r],
                gsem.at[r],
            )

        def store_copy(h, cb):
            r = cb % NRING
            return pltpu.make_async_copy(
                s_v.at[r],
                out_hbm.at[h, :, bc0 + cb, :, :],
                ssem.at[r],
            )

        def fire_gather(h, cb):
            pltpu.async_copy(
                tab_hbm.at[idx_v.at[h, pl.ds(cb * 128, 128)]],
                g_v.at[cb % NRING],
                gsem.at[cb % NRING],
            )

        def transpose_chunk(r):
            def dbody(d, _):
                cols = jnp.full_like(rows16[0], d)
                dd = d // 8
                ds_ = d % 8
                for j2 in range(8):
                    v = plsc.load_gather(g_v.at[r], [rows16[j2], cols])
                    s_v[r, dd, ds_, pl.ds(16 * j2, 16)] = v
                return ()

            lax.fori_loop(0, DIM, dbody, (), unroll=False)

        for cb in range(NRING):
            fire_gather(0, cb)

        def body(h, _):
            for cb in range(CB_N):
                r = cb % NRING
                gather_copy(h, cb).wait()

                @pl.when(h >= 1)
                def _():
                    store_copy(h - 1, cb).wait()

                transpose_chunk(r)

                @pl.when(h + 1 < HIST)
                def _():
                    fire_gather(h + 1, cb)

                pltpu.async_copy(
                    s_v.at[r],
                    out_hbm.at[h, :, bc0 + cb, :, :],
                    ssem.at[r],
                )

            return ()

        lax.fori_loop(0, HIST, body, (), unroll=False)

        for cb in range(CB_N):
            store_copy(HIST - 1, cb).wait()

    return k(tok_t, tab)


def kernel(tokens, emb_table):
    tok_t = tokens.T.astype(jnp.int32)
    out5 = _sc_gather(tok_t, emb_table)
    out = jnp.transpose(out5, (2, 4, 0, 1, 3))
    return out.reshape(BATCH, HIST, DIM)


# speedup vs baseline: 3.0235x; 1.1302x over previous
import functools

import jax
import jax.numpy as jnp
from jax import lax
from jax.experimental import pallas as pl
from jax.experimental.pallas import tpu as pltpu
from jax.experimental.pallas import tpu_sc as plsc

NUM_EMB = 1_000_000
DIM = 32
BATCH = 16384
HIST = 50

NC, NS = 2, 16
NW = NC * NS
BW = BATCH // NW
CB_N = BW // 128
NRING = 4


def _sc_gather(tok_t, tab):
    mesh = plsc.VectorSubcoreMesh(
        core_axis_name="c", subcore_axis_name="s", num_cores=NC, num_subcores=NS
    )

    @functools.partial(
        pl.kernel,
        mesh=mesh,
        out_type=jax.ShapeDtypeStruct((HIST, DIM // 8, BATCH // 128, 8, 128),
                                      jnp.float32),
        scratch_types=[
            pltpu.VMEM((HIST, BW), jnp.int32),
            pltpu.VMEM((NRING, 128, 128), jnp.float32),
            pltpu.VMEM((NRING, DIM // 8, 8, 128), jnp.float32),
            pltpu.SemaphoreType.DMA((NRING,)),
            pltpu.SemaphoreType.DMA((NRING,)),
        ],
        compiler_params=pltpu.CompilerParams(
            use_tc_tiling_on_sc=False, needs_layout_passes=False
        ),
    )
    def k(tok_hbm, tab_hbm, out_hbm, idx_v, g_v, s_v, gsem, ssem):
        c = lax.axis_index("c")
        t = lax.axis_index("s")
        w = t * NC + c
        b0 = w * BW
        bc0 = w * CB_N

        pltpu.sync_copy(tok_hbm.at[:, pl.ds(b0, BW)], idx_v)

        rows16 = [lax.iota(jnp.int32, 16) + 16 * j2 for j2 in range(8)]

        def gather_copy(h, cb):
            r = (cb % NRING)
            return pltpu.make_async_copy(
                tab_hbm.at[idx_v.at[h, pl.ds(cb * 128, 128)]],
                g_v.at[r],
                gsem.at[r],
            )

        def store_copy(h, cb):
            r = cb % NRING
            return pltpu.make_async_copy(
                s_v.at[r],
                out_hbm.at[h, :, bc0 + cb, :, :],
                ssem.at[r],
            )

        def fire_gather(h, cb):
            pltpu.async_copy(
                tab_hbm.at[idx_v.at[h, pl.ds(cb * 128, 128)]],
                g_v.at[cb % NRING],
                gsem.at[cb % NRING],
            )

        def transpose_chunk(r):
            def dbody(d, _):
                cols = jnp.full_like(rows16[0], d)
                dd = d // 8
                ds_ = d % 8
                vs = [
                    plsc.load_gather(g_v.at[r], [rows16[j2], cols])
                    for j2 in range(8)
                ]
                for j2 in range(8):
                    s_v[r, dd, ds_, pl.ds(16 * j2, 16)] = vs[j2]
                return ()

            lax.fori_loop(0, DIM, dbody, (), unroll=False)

        for cb in range(NRING):
            fire_gather(0, cb)

        def body(h, _):
            for cb in range(CB_N):
                r = cb % NRING
                gather_copy(h, cb).wait()

                @pl.when(h >= 1)
                def _():
                    store_copy(h - 1, cb).wait()

                transpose_chunk(r)

                @pl.when(h + 1 < HIST)
                def _():
                    fire_gather(h + 1, cb)

                pltpu.async_copy(
                    s_v.at[r],
                    out_hbm.at[h, :, bc0 + cb, :, :],
                    ssem.at[r],
                )

            return ()

        lax.fori_loop(0, HIST, body, (), unroll=False)

        for cb in range(CB_N):
            store_copy(HIST - 1, cb).wait()

    return k(tok_t, tab)


def kernel(tokens, emb_table):
    tok_t = tokens.T.astype(jnp.int32)
    tab_p = jnp.pad(emb_table, ((0, 0), (0, 128 - DIM)))
    out5 = _sc_gather(tok_t, tab_p)
    out = jnp.transpose(out5, (2, 4, 0, 1, 3))
    return out.reshape(BATCH, HIST, DIM)
